# baseline (speedup 1.0000x reference)
import functools

import jax
import jax.numpy as jnp
from jax import lax
from jax.experimental import pallas as pl
from jax.experimental.pallas import tpu as pltpu
from jax.experimental.pallas import tpu_sc as plsc

B = 512
S = 196
D = 512
P = 512
PL = 5
K = 5
KP = K * PL
ROW = PL * D

NC = 2
NS = 16
NW = NC * NS
IDX_PER_W = (B * K) // NW
CHUNK = 40


def _topk_loss_body(cls_ref, key_ref, idx_ref, loss_ref):
    cls = cls_ref[...]
    key = key_ref[...]
    eps = 1e-12
    kn = key * lax.rsqrt(jnp.maximum(jnp.sum(key * key), eps))
    qn = cls * lax.rsqrt(jnp.maximum(jnp.sum(cls * cls), eps))
    sim = lax.dot_general(qn, kn, (((1,), (1,)), ((), ())))
    cols = lax.broadcasted_iota(jnp.int32, (B, P), 1)
    winners = []
    for _ in range(K):
        m = jnp.max(sim, axis=1, keepdims=True)
        hit = sim == m
        idxk = jnp.min(jnp.where(hit, cols, P), axis=1)
        winners.append(idxk)
        sim = jnp.where(cols == idxk[:, None], -jnp.inf, sim)
    idx_ref[...] = jnp.stack(winners, axis=0)

    loss_ref[...] = jnp.full((1, 1), jnp.sum(kn * qn) / B, jnp.float32)


def _topk_loss(cls_feature, prompt_key):
    return pl.pallas_call(
        _topk_loss_body,
        out_shape=(
            jax.ShapeDtypeStruct((K, B), jnp.int32),
            jax.ShapeDtypeStruct((1, 1), jnp.float32),
        ),
    )(cls_feature, prompt_key)


def _sc_gather_body(prompt_hbm, idx_hbm, out_hbm, idx_v, rows_v, sem):
    wid = lax.axis_index("s") * NC + lax.axis_index("c")
    for c in range(IDX_PER_W // CHUNK):
        base = wid * IDX_PER_W + c * CHUNK
        pltpu.sync_copy(idx_hbm.at[pl.ds(base, CHUNK)], idx_v)
        pltpu.async_copy(prompt_hbm.at[idx_v], rows_v, sem).wait()
        pltpu.sync_copy(rows_v, out_hbm.at[pl.ds(base, CHUNK)])


def _sc_gather(prompt2d, idx_flat):
    mesh = plsc.VectorSubcoreMesh(core_axis_name="c", subcore_axis_name="s")
    return pl.kernel(
        _sc_gather_body,
        out_type=jax.ShapeDtypeStruct((B * K, ROW), jnp.float32),
        mesh=mesh,
        scratch_types=[
            pltpu.VMEM((CHUNK,), jnp.int32),
            pltpu.VMEM((CHUNK, ROW), jnp.float32),
            pltpu.SemaphoreType.DMA,
        ],
    )(prompt2d, idx_flat)


def kernel(x_embed, cls_feature, prompt, prompt_key):
    idx_kb, loss11 = _topk_loss(cls_feature, prompt_key)
    idx_flat = idx_kb.T.reshape(B * K)
    bp = _sc_gather(prompt.reshape(P, ROW), idx_flat)
    res = jnp.concatenate((bp.reshape(B, KP, D), x_embed), axis=1)
    loss = loss11.reshape(())
    return (res, loss)


# speedup vs baseline: 1.0257x; 1.0257x over previous
import functools

import jax
import jax.numpy as jnp
from jax import lax
from jax.experimental import pallas as pl
from jax.experimental.pallas import tpu as pltpu
from jax.experimental.pallas import tpu_sc as plsc

B = 512
S = 196
D = 512
P = 512
PL = 5
K = 5
KP = K * PL
ROW = PL * D

NC = 2
NS = 16
NW = NC * NS
IDX_PER_W = (B * K) // NW
CHUNK = 40


def _topk_loss_body(cls_ref, key_ref, idx_ref, loss_ref):
    cls = cls_ref[...]
    key = key_ref[...]
    eps = 1e-12
    kn = key * lax.rsqrt(jnp.maximum(jnp.sum(key * key), eps))
    qn = cls * lax.rsqrt(jnp.maximum(jnp.sum(cls * cls), eps))
    sim = lax.dot_general(qn, kn, (((1,), (1,)), ((), ())))
    cols = lax.broadcasted_iota(jnp.int32, (B, P), 1)
    winners = []
    for _ in range(K):
        m = jnp.max(sim, axis=1, keepdims=True)
        hit = sim == m
        idxk = jnp.min(jnp.where(hit, cols, P), axis=1)
        winners.append(idxk)
        sim = jnp.where(cols == idxk[:, None], -jnp.inf, sim)
    idx_ref[...] = jnp.stack(winners, axis=0)

    loss_ref[...] = jnp.full((1, 1), jnp.sum(kn * qn) / B, jnp.float32)


def _topk_loss(cls_feature, prompt_key):
    return pl.pallas_call(
        _topk_loss_body,
        out_shape=(
            jax.ShapeDtypeStruct((K, B), jnp.int32),
            jax.ShapeDtypeStruct((1, 1), jnp.float32),
        ),
    )(cls_feature, prompt_key)


def _sc_gather_body(prompt_hbm, idx_hbm, out_hbm, idx_v, rows_v, sem):
    wid = lax.axis_index("s") * NC + lax.axis_index("c")
    for c in range(IDX_PER_W // CHUNK):
        base = wid * IDX_PER_W + c * CHUNK
        pltpu.sync_copy(idx_hbm.at[pl.ds(base, CHUNK)], idx_v)
        pltpu.async_copy(prompt_hbm.at[idx_v], rows_v, sem).wait()
        pltpu.sync_copy(rows_v, out_hbm.at[pl.ds(base, CHUNK)])


def _sc_gather(prompt2d, idx_flat):
    mesh = plsc.VectorSubcoreMesh(core_axis_name="c", subcore_axis_name="s")
    return pl.kernel(
        _sc_gather_body,
        out_type=jax.ShapeDtypeStruct((B * K, ROW), jnp.float32),
        mesh=mesh,
        scratch_types=[
            pltpu.VMEM((CHUNK,), jnp.int32),
            pltpu.VMEM((CHUNK, ROW), jnp.float32),
            pltpu.SemaphoreType.DMA,
        ],
    )(prompt2d, idx_flat)


def kernel(x_embed, cls_feature, prompt, prompt_key):
    idx_kb, loss11 = _topk_loss(cls_feature, prompt_key)
    idx_flat = idx_kb.T.reshape(B * K)
    bp = _sc_gather(prompt.reshape(P, ROW), idx_flat)
    zero_slab = jnp.zeros((B, KP, D), jnp.float32)
    big = jnp.concatenate((zero_slab, x_embed), axis=1)
    res = lax.dynamic_update_slice(big, bp.reshape(B, KP, D), (0, 0, 0))
    loss = loss11.reshape(())
    return (res, loss)
